# baseline (speedup 1.0000x reference)
import functools

import jax
import jax.numpy as jnp
from jax import lax
from jax.experimental import pallas as pl
from jax.experimental.pallas import tpu as pltpu
from jax.experimental.pallas import tpu_sc as plsc

NC = 2
NS = 16
CE = 128
KB = 24
BN = 1000


def _make_deg_kernel(Epad, N, NACC):
    EPW = Epad // (NC * NS)
    nch = EPW // CE
    mesh = plsc.VectorSubcoreMesh(core_axis_name="c", subcore_axis_name="s")

    @functools.partial(
        pl.kernel,
        out_type=jax.ShapeDtypeStruct((NC, NACC, 16), jnp.float32),
        mesh=mesh,
        scratch_types=[
            pltpu.VMEM_SHARED((NACC, 16), jnp.float32),
            pltpu.VMEM((1, CE), jnp.int32),
            pltpu.VMEM((CE, 16), jnp.float32),
        ],
    )
    def deg_k(dst_hbm, ones_hbm, zer_hbm, out_hbm, acc_sh, didx, ones_v):
        c = lax.axis_index("c")
        s = lax.axis_index("s")
        pltpu.sync_copy(ones_hbm, ones_v)
        zr = NACC // NS
        pltpu.sync_copy(zer_hbm.at[pl.ds(s * zr, zr)],
                        acc_sh.at[pl.ds(s * zr, zr)])
        plsc.subcore_barrier()
        base = (c * NS + s) * EPW

        def body(k, carry):
            off = pl.multiple_of(base + k * CE, CE)
            pltpu.sync_copy(dst_hbm.at[pl.ds(off, CE)], didx.at[0])
            pltpu.sync_copy(ones_v, acc_sh.at[didx.at[0]], add=True)
            return carry

        lax.fori_loop(0, nch, body, 0)
        plsc.subcore_barrier()
        orr = NACC // NS
        pltpu.sync_copy(acc_sh.at[pl.ds(s * orr, orr)],
                        out_hbm.at[c].at[pl.ds(s * orr, orr)])

    return deg_k


def _make_prop_kernel(Epad, N, NACC, F):
    EPT = Epad // NS
    nch = EPT // CE
    nblk = nch // KB
    assert nch % KB == 0 and KB % 2 == 0
    mesh = plsc.VectorSubcoreMesh(core_axis_name="c", subcore_axis_name="s")

    @functools.partial(
        pl.kernel,
        out_type=jax.ShapeDtypeStruct((NC, NACC, F), jnp.float32),
        mesh=mesh,
        scratch_types=[
            pltpu.VMEM_SHARED((NACC, F), jnp.float32),
            pltpu.VMEM((2, KB, CE), jnp.int32),
            pltpu.VMEM((2, KB, CE), jnp.int32),
            pltpu.VMEM((CE, F), jnp.float32),
            pltpu.VMEM((CE, F), jnp.float32),
            pltpu.SemaphoreType.DMA,
            pltpu.SemaphoreType.DMA,
        ],
    )
    def prop_k(g_hbm, src2_hbm, dst_hbm, zer_hbm, out_hbm,
               acc_sh, sidx, didx, rows0, rows1, sem0, sem1):
        c = lax.axis_index("c")
        s = lax.axis_index("s")
        pltpu.sync_copy(src2_hbm.at[c].at[s].at[pl.ds(0, KB)], sidx.at[0])
        pltpu.sync_copy(dst_hbm.at[s].at[pl.ds(0, KB)], didx.at[0])
        zr = NACC // NS
        pltpu.sync_copy(zer_hbm.at[pl.ds(s * zr, zr)],
                        acc_sh.at[pl.ds(s * zr, zr)])
        plsc.subcore_barrier()

        rows = (rows0, rows1)
        sems = (sem0, sem1)
        pltpu.async_copy(g_hbm.at[sidx.at[0].at[0]], rows0, sem0)

        def body(j, carry):
            slot = lax.rem(j, 2)
            nslot = 1 - slot
            nxt = pl.multiple_of(
                jnp.where(j + 1 < nblk, (j + 1) * KB, 0), 8)
            pltpu.sync_copy(src2_hbm.at[c].at[s].at[pl.ds(nxt, KB)],
                            sidx.at[nslot])
            pltpu.sync_copy(dst_hbm.at[s].at[pl.ds(nxt, KB)],
                            didx.at[nslot])
            for t in range(KB):
                b = t % 2
                pltpu.make_async_copy(g_hbm.at[sidx.at[0].at[0]],
                                      rows[b], sems[b]).wait()
                if t + 1 < KB:
                    nidx = sidx.at[slot].at[t + 1]
                else:
                    nidx = sidx.at[nslot].at[0]
                pltpu.async_copy(g_hbm.at[nidx], rows[1 - b], sems[1 - b])
                pltpu.sync_copy(rows[b], acc_sh.at[didx.at[slot].at[t]],
                                add=True)
            return carry

        lax.fori_loop(0, nblk, body, 0)
        pltpu.make_async_copy(g_hbm.at[sidx.at[0].at[0]], rows0, sem0).wait()
        plsc.subcore_barrier()
        orr = NACC // NS
        pltpu.sync_copy(acc_sh.at[pl.ds(s * orr, orr)],
                        out_hbm.at[c].at[pl.ds(s * orr, orr)])

    return prop_k


def _dinv_block(deg_ref):
    deg3 = deg_ref[...]
    return lax.rsqrt(deg3[0] + deg3[1])[:, 0:1]


def _k0_body(x_ref, w_ref, deg_ref, g_ref):
    F = g_ref.shape[2]
    dinv = _dinv_block(deg_ref)
    h = jnp.dot(x_ref[...], w_ref[...], preferred_element_type=jnp.float32)
    g = h * dinv
    g_ref[0] = g[:, :F]
    g_ref[1] = g[:, F:]


def _k12_body(acc_ref, deg_ref, b_ref, w_ref, x_ref, g_ref):
    F = g_ref.shape[2]
    dinv = _dinv_block(deg_ref)
    acc = jnp.concatenate([acc_ref[0], acc_ref[1]], axis=1)
    xl = jnp.maximum(acc * dinv + b_ref[...], 0.0)
    x_ref[...] = xl
    g = jnp.dot(xl, w_ref[...], preferred_element_type=jnp.float32) * dinv
    g_ref[0] = g[:, :F]
    g_ref[1] = g[:, F:]


def _k3_body(acc_ref, deg_ref, b_ref, x1_ref, x2_ref, wl_ref, bl_ref, out_ref):
    H = x1_ref.shape[1]
    dinv = _dinv_block(deg_ref)
    x3 = jnp.maximum(
        jnp.concatenate([acc_ref[0], acc_ref[1]], axis=1) * dinv + b_ref[...],
        0.0)
    p = jnp.dot(x1_ref[...], wl_ref[0:H, :], preferred_element_type=jnp.float32)
    p = p + jnp.dot(x2_ref[...], wl_ref[H:2 * H, :],
                    preferred_element_type=jnp.float32)
    p = p + jnp.dot(x3, wl_ref[2 * H:3 * H, :],
                    preferred_element_type=jnp.float32)
    out_ref[...] = p + bl_ref[...]


def _row_spec(shape2):
    if len(shape2) == 2:
        return pl.BlockSpec((BN, shape2[1]), lambda i: (i, 0))
    return pl.BlockSpec((shape2[0], BN, shape2[2]), lambda i: (0, i, 0))


def _full_spec(shape):
    nd = len(shape)
    return pl.BlockSpec(shape, lambda i: (0,) * nd)


def kernel(x, edge_index, W0, b0, W1, b1, W2, b2, Wlin, blin):
    N, D = x.shape
    H = W0.shape[1]
    OUT = Wlin.shape[1]
    E = edge_index.shape[1]
    F = H // 2
    Etot = E + N

    EPT = -(-Etot // (NS * KB * CE)) * (KB * CE)
    Epad = NS * EPT
    NACC = -(-(N + 1) // (NS * 8)) * (NS * 8)

    ei = edge_index.astype(jnp.int32)
    loop = jnp.arange(N, dtype=jnp.int32)
    src = jnp.concatenate([ei[0], loop])
    dst = jnp.concatenate([ei[1], loop])
    pad = Epad - Etot
    src_p = jnp.concatenate([src, jnp.zeros((pad,), jnp.int32)])
    dst_p = jnp.concatenate([dst, jnp.full((pad,), N, jnp.int32)])
    src2 = jnp.concatenate([src_p, src_p + N])
    nch = Epad // (NS * CE)
    src2_t = src2.reshape(NC, NS, nch, CE)
    dst_t = dst_p.reshape(NS, nch, CE)

    ones16 = jnp.ones((CE, 16), jnp.float32)
    zer_d = jnp.zeros((NACC, 16), jnp.float32)
    zer_f = jnp.zeros((NACC, F), jnp.float32)
    b0r = b0.reshape(1, H)
    b1r = b1.reshape(1, H)
    b2r = b2.reshape(1, H)
    blr = blin.reshape(1, OUT)

    deg_k = _make_deg_kernel(Epad, N, NACC)
    prop_k = _make_prop_kernel(Epad, N, NACC, F)
    grid = (N // BN,)

    deg2 = deg_k(dst_p, ones16, zer_d)

    g0 = pl.pallas_call(
        _k0_body,
        grid=grid,
        in_specs=[_row_spec((N, D)), _full_spec((D, H)),
                  _row_spec((NC, N, 16))],
        out_specs=_row_spec((NC, N, F)),
        out_shape=jax.ShapeDtypeStruct((NC, N, F), jnp.float32),
    )(x, W0, deg2)

    acc1 = prop_k(g0.reshape(NC * N, F), src2_t, dst_t, zer_f)

    def layer(acc, b_r, W_next):
        return pl.pallas_call(
            _k12_body,
            grid=grid,
            in_specs=[_row_spec((NC, N, F)), _row_spec((NC, N, 16)),
                      _full_spec((1, H)), _full_spec((H, H))],
            out_specs=[_row_spec((N, H)), _row_spec((NC, N, F))],
            out_shape=[jax.ShapeDtypeStruct((N, H), jnp.float32),
                       jax.ShapeDtypeStruct((NC, N, F), jnp.float32)],
        )(acc, deg2, b_r, W_next)

    x1, g1 = layer(acc1, b0r, W1)
    acc2 = prop_k(g1.reshape(NC * N, F), src2_t, dst_t, zer_f)
    x2, g2 = layer(acc2, b1r, W2)
    acc3 = prop_k(g2.reshape(NC * N, F), src2_t, dst_t, zer_f)

    pred = pl.pallas_call(
        _k3_body,
        grid=grid,
        in_specs=[_row_spec((NC, N, F)), _row_spec((NC, N, 16)),
                  _full_spec((1, H)), _row_spec((N, H)), _row_spec((N, H)),
                  _full_spec((3 * H, OUT)), _full_spec((1, OUT))],
        out_specs=_row_spec((N, OUT)),
        out_shape=jax.ShapeDtypeStruct((N, OUT), jnp.float32),
    )(acc3, deg2, b2r, x1, x2, Wlin, blr)

    return pred


# speedup vs baseline: 2.3667x; 2.3667x over previous
import functools

import jax
import jax.numpy as jnp
from jax import lax
from jax.experimental import pallas as pl
from jax.experimental.pallas import tpu as pltpu
from jax.experimental.pallas import tpu_sc as plsc

NC = 2
NS = 16
CE = 128
KB = 24
BN = 1000


def _make_deg_kernel(Epad, N, NACC):
    EPW = Epad // (NC * NS)
    nch = EPW // CE
    mesh = plsc.VectorSubcoreMesh(core_axis_name="c", subcore_axis_name="s")

    @functools.partial(
        pl.kernel,
        out_type=jax.ShapeDtypeStruct((NC, NACC, 16), jnp.float32),
        mesh=mesh,
        scratch_types=[
            pltpu.VMEM_SHARED((NACC, 16), jnp.float32),
            pltpu.VMEM((1, CE), jnp.int32),
            pltpu.VMEM((CE, 16), jnp.float32),
        ],
    )
    def deg_k(dst_hbm, ones_hbm, zer_hbm, out_hbm, acc_sh, didx, ones_v):
        c = lax.axis_index("c")
        s = lax.axis_index("s")
        pltpu.sync_copy(ones_hbm, ones_v)
        zr = NACC // NS
        pltpu.sync_copy(zer_hbm.at[pl.ds(s * zr, zr)],
                        acc_sh.at[pl.ds(s * zr, zr)])
        plsc.subcore_barrier()
        base = (c * NS + s) * EPW

        def body(k, carry):
            off = pl.multiple_of(base + k * CE, CE)
            pltpu.sync_copy(dst_hbm.at[pl.ds(off, CE)], didx.at[0])
            pltpu.sync_copy(ones_v, acc_sh.at[didx.at[0]], add=True)
            return carry

        lax.fori_loop(0, nch, body, 0)
        plsc.subcore_barrier()
        orr = NACC // NS
        pltpu.sync_copy(acc_sh.at[pl.ds(s * orr, orr)],
                        out_hbm.at[c].at[pl.ds(s * orr, orr)])

    return deg_k


def _make_prop_kernel(Epad, N, NACC, F):
    EPT = Epad // NS
    nch = EPT // CE
    assert nch % 2 == 0
    mesh = plsc.VectorSubcoreMesh(core_axis_name="c", subcore_axis_name="s")

    @functools.partial(
        pl.kernel,
        out_type=jax.ShapeDtypeStruct((NC, NACC, F), jnp.float32),
        mesh=mesh,
        scratch_types=[
            pltpu.VMEM_SHARED((NACC, F), jnp.float32),
            pltpu.VMEM((1, CE), jnp.int32),
            pltpu.VMEM((1, CE), jnp.int32),
            pltpu.VMEM((1, CE), jnp.int32),
            pltpu.VMEM((1, CE), jnp.int32),
            pltpu.VMEM((CE, F), jnp.float32),
            pltpu.VMEM((CE, F), jnp.float32),
            pltpu.SemaphoreType.DMA,
            pltpu.SemaphoreType.DMA,
        ],
    )
    def prop_k(g_hbm, src2_hbm, dst_hbm, zer_hbm, out_hbm,
               acc_sh, sidx0, didx0, sidx1, didx1, rows0, rows1, sem0, sem1):
        c = lax.axis_index("c")
        s = lax.axis_index("s")
        zr = NACC // NS
        pltpu.sync_copy(zer_hbm.at[pl.ds(s * zr, zr)],
                        acc_sh.at[pl.ds(s * zr, zr)])
        plsc.subcore_barrier()
        sbase = c * Epad + s * EPT
        dbase = s * EPT

        def body(i, carry):
            o0 = pl.multiple_of(2 * i * CE, 2 * CE)
            pltpu.sync_copy(src2_hbm.at[pl.ds(sbase + o0, CE)], sidx0.at[0])
            pltpu.sync_copy(dst_hbm.at[pl.ds(dbase + o0, CE)], didx0.at[0])
            h0 = pltpu.async_copy(g_hbm.at[sidx0.at[0]], rows0, sem0)
            pltpu.sync_copy(src2_hbm.at[pl.ds(sbase + o0 + CE, CE)],
                            sidx1.at[0])
            pltpu.sync_copy(dst_hbm.at[pl.ds(dbase + o0 + CE, CE)],
                            didx1.at[0])
            h1 = pltpu.async_copy(g_hbm.at[sidx1.at[0]], rows1, sem1)
            h0.wait()
            pltpu.sync_copy(rows0, acc_sh.at[didx0.at[0]], add=True)
            h1.wait()
            pltpu.sync_copy(rows1, acc_sh.at[didx1.at[0]], add=True)
            return carry

        lax.fori_loop(0, nch // 2, body, 0)
        plsc.subcore_barrier()
        orr = NACC // NS
        pltpu.sync_copy(acc_sh.at[pl.ds(s * orr, orr)],
                        out_hbm.at[c].at[pl.ds(s * orr, orr)])

    return prop_k


def _dinv_block(deg_ref):
    deg3 = deg_ref[...]
    return lax.rsqrt(deg3[0] + deg3[1])[:, 0:1]


def _k0_body(x_ref, w_ref, deg_ref, g_ref):
    F = g_ref.shape[2]
    dinv = _dinv_block(deg_ref)
    h = jnp.dot(x_ref[...], w_ref[...], preferred_element_type=jnp.float32)
    g = h * dinv
    g_ref[0] = g[:, :F]
    g_ref[1] = g[:, F:]


def _k12_body(acc_ref, deg_ref, b_ref, w_ref, x_ref, g_ref):
    F = g_ref.shape[2]
    dinv = _dinv_block(deg_ref)
    acc = jnp.concatenate([acc_ref[0], acc_ref[1]], axis=1)
    xl = jnp.maximum(acc * dinv + b_ref[...], 0.0)
    x_ref[...] = xl
    g = jnp.dot(xl, w_ref[...], preferred_element_type=jnp.float32) * dinv
    g_ref[0] = g[:, :F]
    g_ref[1] = g[:, F:]


def _k3_body(acc_ref, deg_ref, b_ref, x1_ref, x2_ref, wl_ref, bl_ref, out_ref):
    H = x1_ref.shape[1]
    dinv = _dinv_block(deg_ref)
    x3 = jnp.maximum(
        jnp.concatenate([acc_ref[0], acc_ref[1]], axis=1) * dinv + b_ref[...],
        0.0)
    p = jnp.dot(x1_ref[...], wl_ref[0:H, :], preferred_element_type=jnp.float32)
    p = p + jnp.dot(x2_ref[...], wl_ref[H:2 * H, :],
                    preferred_element_type=jnp.float32)
    p = p + jnp.dot(x3, wl_ref[2 * H:3 * H, :],
                    preferred_element_type=jnp.float32)
    out_ref[...] = p + bl_ref[...]


def _row_spec(shape2):
    if len(shape2) == 2:
        return pl.BlockSpec((BN, shape2[1]), lambda i: (i, 0))
    return pl.BlockSpec((shape2[0], BN, shape2[2]), lambda i: (0, i, 0))


def _full_spec(shape):
    nd = len(shape)
    return pl.BlockSpec(shape, lambda i: (0,) * nd)


def kernel(x, edge_index, W0, b0, W1, b1, W2, b2, Wlin, blin):
    N, D = x.shape
    H = W0.shape[1]
    OUT = Wlin.shape[1]
    E = edge_index.shape[1]
    F = H // 2
    Etot = E + N

    EPT = -(-Etot // (NS * 2 * CE)) * (2 * CE)
    Epad = NS * EPT
    NACC = -(-(N + 1) // (NS * 8)) * (NS * 8)

    ei = edge_index.astype(jnp.int32)
    loop = jnp.arange(N, dtype=jnp.int32)
    src = jnp.concatenate([ei[0], loop])
    dst = jnp.concatenate([ei[1], loop])
    pad = Epad - Etot
    src_p = jnp.concatenate([src, jnp.zeros((pad,), jnp.int32)])
    dst_p = jnp.concatenate([dst, jnp.full((pad,), N, jnp.int32)])
    src2 = jnp.concatenate([src_p, src_p + N])

    ones16 = jnp.ones((CE, 16), jnp.float32)
    zer_d = jnp.zeros((NACC, 16), jnp.float32)
    zer_f = jnp.zeros((NACC, F), jnp.float32)
    b0r = b0.reshape(1, H)
    b1r = b1.reshape(1, H)
    b2r = b2.reshape(1, H)
    blr = blin.reshape(1, OUT)

    deg_k = _make_deg_kernel(Epad, N, NACC)
    prop_k = _make_prop_kernel(Epad, N, NACC, F)
    grid = (N // BN,)

    deg2 = deg_k(dst_p, ones16, zer_d)

    g0 = pl.pallas_call(
        _k0_body,
        grid=grid,
        in_specs=[_row_spec((N, D)), _full_spec((D, H)),
                  _row_spec((NC, N, 16))],
        out_specs=_row_spec((NC, N, F)),
        out_shape=jax.ShapeDtypeStruct((NC, N, F), jnp.float32),
    )(x, W0, deg2)

    acc1 = prop_k(g0.reshape(NC * N, F), src2, dst_p, zer_f)

    def layer(acc, b_r, W_next):
        return pl.pallas_call(
            _k12_body,
            grid=grid,
            in_specs=[_row_spec((NC, N, F)), _row_spec((NC, N, 16)),
                      _full_spec((1, H)), _full_spec((H, H))],
            out_specs=[_row_spec((N, H)), _row_spec((NC, N, F))],
            out_shape=[jax.ShapeDtypeStruct((N, H), jnp.float32),
                       jax.ShapeDtypeStruct((NC, N, F), jnp.float32)],
        )(acc, deg2, b_r, W_next)

    x1, g1 = layer(acc1, b0r, W1)
    acc2 = prop_k(g1.reshape(NC * N, F), src2, dst_p, zer_f)
    x2, g2 = layer(acc2, b1r, W2)
    acc3 = prop_k(g2.reshape(NC * N, F), src2, dst_p, zer_f)

    pred = pl.pallas_call(
        _k3_body,
        grid=grid,
        in_specs=[_row_spec((NC, N, F)), _row_spec((NC, N, 16)),
                  _full_spec((1, H)), _row_spec((N, H)), _row_spec((N, H)),
                  _full_spec((3 * H, OUT)), _full_spec((1, OUT))],
        out_specs=_row_spec((N, OUT)),
        out_shape=jax.ShapeDtypeStruct((N, OUT), jnp.float32),
    )(acc3, deg2, b2r, x1, x2, Wlin, blr)

    return pred


# speedup vs baseline: 3.0817x; 1.3021x over previous
import functools

import jax
import jax.numpy as jnp
from jax import lax
from jax.experimental import pallas as pl
from jax.experimental.pallas import tpu as pltpu
from jax.experimental.pallas import tpu_sc as plsc

NC = 2
NS = 16
CE = 128
KB = 24
BN = 1000


def _make_deg_kernel(Epad, N, NACC):
    EPW = Epad // (NC * NS)
    nch = EPW // CE
    mesh = plsc.VectorSubcoreMesh(core_axis_name="c", subcore_axis_name="s")

    @functools.partial(
        pl.kernel,
        out_type=jax.ShapeDtypeStruct((NC, NACC, 16), jnp.float32),
        mesh=mesh,
        scratch_types=[
            pltpu.VMEM_SHARED((NACC, 16), jnp.float32),
            pltpu.VMEM((1, CE), jnp.int32),
            pltpu.VMEM((CE, 16), jnp.float32),
        ],
    )
    def deg_k(dst_hbm, ones_hbm, zer_hbm, out_hbm, acc_sh, didx, ones_v):
        c = lax.axis_index("c")
        s = lax.axis_index("s")
        pltpu.sync_copy(ones_hbm, ones_v)
        zr = NACC // NS
        pltpu.sync_copy(zer_hbm.at[pl.ds(s * zr, zr)],
                        acc_sh.at[pl.ds(s * zr, zr)])
        plsc.subcore_barrier()
        base = (c * NS + s) * EPW

        def body(k, carry):
            off = pl.multiple_of(base + k * CE, CE)
            pltpu.sync_copy(dst_hbm.at[pl.ds(off, CE)], didx.at[0])
            pltpu.sync_copy(ones_v, acc_sh.at[didx.at[0]], add=True)
            return carry

        lax.fori_loop(0, nch, body, 0)
        plsc.subcore_barrier()
        orr = NACC // NS
        pltpu.sync_copy(acc_sh.at[pl.ds(s * orr, orr)],
                        out_hbm.at[c].at[pl.ds(s * orr, orr)])

    return deg_k


def _make_prop_kernel(Epad, N, NACC, F):
    EPT = Epad // NS
    nch = EPT // CE
    assert nch % 2 == 0
    mesh = plsc.VectorSubcoreMesh(core_axis_name="c", subcore_axis_name="s")

    @functools.partial(
        pl.kernel,
        out_type=jax.ShapeDtypeStruct((NC, NACC, F), jnp.float32),
        mesh=mesh,
        scratch_types=[
            pltpu.VMEM_SHARED((NACC, F), jnp.float32),
            pltpu.VMEM((2, CE), jnp.int32),
            pltpu.VMEM((2, CE), jnp.int32),
            pltpu.VMEM((CE, F), jnp.float32),
            pltpu.VMEM((CE, F), jnp.float32),
            pltpu.SemaphoreType.DMA,
            pltpu.SemaphoreType.DMA,
        ],
    )
    def prop_k(g_hbm, sd_hbm, zer_hbm, out_hbm,
               acc_sh, idx0, idx1, rows0, rows1, sem0, sem1):
        c = lax.axis_index("c")
        s = lax.axis_index("s")
        zr = NACC // NS
        pltpu.sync_copy(zer_hbm.at[pl.ds(s * zr, zr)],
                        acc_sh.at[pl.ds(s * zr, zr)])
        plsc.subcore_barrier()
        my_sd = sd_hbm.at[c].at[s]

        pltpu.sync_copy(my_sd.at[0], idx0)
        pltpu.async_copy(g_hbm.at[idx0.at[0]], rows0, sem0)

        def body(i, carry):
            pltpu.sync_copy(my_sd.at[2 * i + 1], idx1)
            pltpu.async_copy(g_hbm.at[idx1.at[0]], rows1, sem1)
            pltpu.make_async_copy(g_hbm.at[idx0.at[0]], rows0, sem0).wait()
            pltpu.sync_copy(rows0, acc_sh.at[idx0.at[1]], add=True)
            pltpu.sync_copy(my_sd.at[jnp.minimum(2 * i + 2, nch - 1)], idx0)
            pltpu.async_copy(g_hbm.at[idx0.at[0]], rows0, sem0)
            pltpu.make_async_copy(g_hbm.at[idx1.at[0]], rows1, sem1).wait()
            pltpu.sync_copy(rows1, acc_sh.at[idx1.at[1]], add=True)
            return carry

        lax.fori_loop(0, nch // 2, body, 0)
        pltpu.make_async_copy(g_hbm.at[idx0.at[0]], rows0, sem0).wait()
        plsc.subcore_barrier()
        orr = NACC // NS
        pltpu.sync_copy(acc_sh.at[pl.ds(s * orr, orr)],
                        out_hbm.at[c].at[pl.ds(s * orr, orr)])

    return prop_k


def _dinv_block(deg_ref):
    deg3 = deg_ref[...]
    return lax.rsqrt(deg3[0] + deg3[1])[:, 0:1]


def _k0_body(x_ref, w_ref, deg_ref, g_ref):
    F = g_ref.shape[2]
    dinv = _dinv_block(deg_ref)
    h = jnp.dot(x_ref[...], w_ref[...], preferred_element_type=jnp.float32)
    g = h * dinv
    g_ref[0] = g[:, :F]
    g_ref[1] = g[:, F:]


def _k12_body(acc_ref, deg_ref, b_ref, w_ref, x_ref, g_ref):
    F = g_ref.shape[2]
    dinv = _dinv_block(deg_ref)
    acc = jnp.concatenate([acc_ref[0], acc_ref[1]], axis=1)
    xl = jnp.maximum(acc * dinv + b_ref[...], 0.0)
    x_ref[...] = xl
    g = jnp.dot(xl, w_ref[...], preferred_element_type=jnp.float32) * dinv
    g_ref[0] = g[:, :F]
    g_ref[1] = g[:, F:]


def _k3_body(acc_ref, deg_ref, b_ref, x1_ref, x2_ref, wl_ref, bl_ref, out_ref):
    H = x1_ref.shape[1]
    dinv = _dinv_block(deg_ref)
    x3 = jnp.maximum(
        jnp.concatenate([acc_ref[0], acc_ref[1]], axis=1) * dinv + b_ref[...],
        0.0)
    p = jnp.dot(x1_ref[...], wl_ref[0:H, :], preferred_element_type=jnp.float32)
    p = p + jnp.dot(x2_ref[...], wl_ref[H:2 * H, :],
                    preferred_element_type=jnp.float32)
    p = p + jnp.dot(x3, wl_ref[2 * H:3 * H, :],
                    preferred_element_type=jnp.float32)
    out_ref[...] = p + bl_ref[...]


def _row_spec(shape2):
    if len(shape2) == 2:
        return pl.BlockSpec((BN, shape2[1]), lambda i: (i, 0))
    return pl.BlockSpec((shape2[0], BN, shape2[2]), lambda i: (0, i, 0))


def _full_spec(shape):
    nd = len(shape)
    return pl.BlockSpec(shape, lambda i: (0,) * nd)


def kernel(x, edge_index, W0, b0, W1, b1, W2, b2, Wlin, blin):
    N, D = x.shape
    H = W0.shape[1]
    OUT = Wlin.shape[1]
    E = edge_index.shape[1]
    F = H // 2
    Etot = E + N

    EPT = -(-Etot // (NS * 2 * CE)) * (2 * CE)
    Epad = NS * EPT
    NACC = -(-(N + 1) // (NS * 8)) * (NS * 8)

    ei = edge_index.astype(jnp.int32)
    loop = jnp.arange(N, dtype=jnp.int32)
    src = jnp.concatenate([ei[0], loop])
    dst = jnp.concatenate([ei[1], loop])
    pad = Epad - Etot
    src_p = jnp.concatenate([src, jnp.zeros((pad,), jnp.int32)])
    dst_p = jnp.concatenate([dst, jnp.full((pad,), N, jnp.int32)])
    src2 = jnp.concatenate([src_p, src_p + N])
    nch = Epad // (NS * CE)
    sd = jnp.stack([
        src2.reshape(NC, NS, nch, CE),
        jnp.broadcast_to(dst_p.reshape(1, NS, nch, CE), (NC, NS, nch, CE)),
    ], axis=3)

    ones16 = jnp.ones((CE, 16), jnp.float32)
    zer_d = jnp.zeros((NACC, 16), jnp.float32)
    zer_f = jnp.zeros((NACC, F), jnp.float32)
    b0r = b0.reshape(1, H)
    b1r = b1.reshape(1, H)
    b2r = b2.reshape(1, H)
    blr = blin.reshape(1, OUT)

    deg_k = _make_deg_kernel(Epad, N, NACC)
    prop_k = _make_prop_kernel(Epad, N, NACC, F)
    grid = (N // BN,)

    deg2 = deg_k(dst_p, ones16, zer_d)

    g0 = pl.pallas_call(
        _k0_body,
        grid=grid,
        in_specs=[_row_spec((N, D)), _full_spec((D, H)),
                  _row_spec((NC, N, 16))],
        out_specs=_row_spec((NC, N, F)),
        out_shape=jax.ShapeDtypeStruct((NC, N, F), jnp.float32),
    )(x, W0, deg2)

    acc1 = prop_k(g0.reshape(NC * N, F), sd, zer_f)

    def layer(acc, b_r, W_next):
        return pl.pallas_call(
            _k12_body,
            grid=grid,
            in_specs=[_row_spec((NC, N, F)), _row_spec((NC, N, 16)),
                      _full_spec((1, H)), _full_spec((H, H))],
            out_specs=[_row_spec((N, H)), _row_spec((NC, N, F))],
            out_shape=[jax.ShapeDtypeStruct((N, H), jnp.float32),
                       jax.ShapeDtypeStruct((NC, N, F), jnp.float32)],
        )(acc, deg2, b_r, W_next)

    x1, g1 = layer(acc1, b0r, W1)
    acc2 = prop_k(g1.reshape(NC * N, F), sd, zer_f)
    x2, g2 = layer(acc2, b1r, W2)
    acc3 = prop_k(g2.reshape(NC * N, F), sd, zer_f)

    pred = pl.pallas_call(
        _k3_body,
        grid=grid,
        in_specs=[_row_spec((NC, N, F)), _row_spec((NC, N, 16)),
                  _full_spec((1, H)), _row_spec((N, H)), _row_spec((N, H)),
                  _full_spec((3 * H, OUT)), _full_spec((1, OUT))],
        out_specs=_row_spec((N, OUT)),
        out_shape=jax.ShapeDtypeStruct((N, OUT), jnp.float32),
    )(acc3, deg2, b2r, x1, x2, Wlin, blr)

    return pred


# speedup vs baseline: 3.1826x; 1.0328x over previous
import functools

import jax
import jax.numpy as jnp
from jax import lax
from jax.experimental import pallas as pl
from jax.experimental.pallas import tpu as pltpu
from jax.experimental.pallas import tpu_sc as plsc

NC = 2
NS = 16
CE = 96
BN = 1000


def _make_deg_kernel(Epad, N, NACC):
    EPW = Epad // (NC * NS)
    nch = EPW // CE
    mesh = plsc.VectorSubcoreMesh(core_axis_name="c", subcore_axis_name="s")

    @functools.partial(
        pl.kernel,
        out_type=jax.ShapeDtypeStruct((NC, NACC, 16), jnp.float32),
        mesh=mesh,
        scratch_types=[
            pltpu.VMEM_SHARED((NACC, 16), jnp.float32),
            pltpu.VMEM((1, CE), jnp.int32),
            pltpu.VMEM((CE, 16), jnp.float32),
        ],
    )
    def deg_k(dst_hbm, ones_hbm, zer_hbm, out_hbm, acc_sh, didx, ones_v):
        c = lax.axis_index("c")
        s = lax.axis_index("s")
        pltpu.sync_copy(ones_hbm, ones_v)
        zr = NACC // NS
        pltpu.sync_copy(zer_hbm.at[pl.ds(s * zr, zr)],
                        acc_sh.at[pl.ds(s * zr, zr)])
        plsc.subcore_barrier()
        base = (c * NS + s) * EPW

        def body(k, carry):
            off = pl.multiple_of(base + k * CE, CE)
            pltpu.sync_copy(dst_hbm.at[pl.ds(off, CE)], didx.at[0])
            pltpu.sync_copy(ones_v, acc_sh.at[didx.at[0]], add=True)
            return carry

        lax.fori_loop(0, nch, body, 0)
        plsc.subcore_barrier()
        orr = NACC // NS
        pltpu.sync_copy(acc_sh.at[pl.ds(s * orr, orr)],
                        out_hbm.at[c].at[pl.ds(s * orr, orr)])

    return deg_k


def _make_prop_kernel(Epad, N, NACC, F):
    EPT = Epad // NS
    nch = EPT // CE
    assert nch % 4 == 0
    mesh = plsc.VectorSubcoreMesh(core_axis_name="c", subcore_axis_name="s")

    @functools.partial(
        pl.kernel,
        out_type=jax.ShapeDtypeStruct((NC, NACC, F), jnp.float32),
        mesh=mesh,
        scratch_types=[
            pltpu.VMEM_SHARED((NACC, F), jnp.float32),
            [pltpu.VMEM((2, CE), jnp.int32)] * 4,
            [pltpu.VMEM((CE, F), jnp.float32)] * 4,
            [pltpu.SemaphoreType.DMA] * 4,
            [pltpu.SemaphoreType.DMA] * 4,
        ],
    )
    def prop_k(g_hbm, sd_hbm, zer_hbm, out_hbm,
               acc_sh, idx, rows, gsem, ssem):
        c = lax.axis_index("c")
        s = lax.axis_index("s")
        zr = NACC // NS
        pltpu.sync_copy(zer_hbm.at[pl.ds(s * zr, zr)],
                        acc_sh.at[pl.ds(s * zr, zr)])
        plsc.subcore_barrier()
        my_sd = sd_hbm.at[c].at[s]

        pltpu.sync_copy(my_sd.at[0], idx[0])
        pltpu.async_copy(g_hbm.at[idx[0].at[0]], rows[0], gsem[0])
        pltpu.sync_copy(my_sd.at[1], idx[1])
        pltpu.async_copy(g_hbm.at[idx[1].at[0]], rows[1], gsem[1])

        def body(i, carry):
            for b in range(4):
                k = 4 * i + b
                s2 = (b + 2) % 4
                pltpu.make_async_copy(g_hbm.at[idx[b].at[0]],
                                      rows[b], gsem[b]).wait()
                pltpu.async_copy(rows[b], acc_sh.at[idx[b].at[1]],
                                 ssem[b], add=True)
                retire = pltpu.make_async_copy(
                    zer_hbm.at[pl.ds(0, CE)], rows[s2], ssem[s2])
                if b < 2:
                    @pl.when(i > 0)
                    def _():
                        retire.wait()
                else:
                    retire.wait()
                pltpu.sync_copy(my_sd.at[jnp.minimum(k + 2, nch - 1)],
                                idx[s2])
                pltpu.async_copy(g_hbm.at[idx[s2].at[0]], rows[s2], gsem[s2])
            return carry

        lax.fori_loop(0, nch // 4, body, 0)
        pltpu.make_async_copy(g_hbm.at[idx[0].at[0]], rows[0], gsem[0]).wait()
        pltpu.make_async_copy(g_hbm.at[idx[1].at[0]], rows[1], gsem[1]).wait()
        pltpu.make_async_copy(zer_hbm.at[pl.ds(0, CE)], rows[2],
                              ssem[2]).wait()
        pltpu.make_async_copy(zer_hbm.at[pl.ds(0, CE)], rows[3],
                              ssem[3]).wait()
        plsc.subcore_barrier()
        orr = NACC // NS
        pltpu.sync_copy(acc_sh.at[pl.ds(s * orr, orr)],
                        out_hbm.at[c].at[pl.ds(s * orr, orr)])

    return prop_k


def _dinv_block(deg_ref):
    deg3 = deg_ref[...]
    return lax.rsqrt(deg3[0] + deg3[1])[:, 0:1]


def _k0_body(x_ref, w_ref, deg_ref, g_ref):
    F = g_ref.shape[2]
    dinv = _dinv_block(deg_ref)
    h = jnp.dot(x_ref[...], w_ref[...], preferred_element_type=jnp.float32)
    g = h * dinv
    g_ref[0] = g[:, :F]
    g_ref[1] = g[:, F:]


def _k12_body(acc_ref, deg_ref, b_ref, w_ref, x_ref, g_ref):
    F = g_ref.shape[2]
    dinv = _dinv_block(deg_ref)
    acc = jnp.concatenate([acc_ref[0], acc_ref[1]], axis=1)
    xl = jnp.maximum(acc * dinv + b_ref[...], 0.0)
    x_ref[...] = xl
    g = jnp.dot(xl, w_ref[...], preferred_element_type=jnp.float32) * dinv
    g_ref[0] = g[:, :F]
    g_ref[1] = g[:, F:]


def _k3_body(acc_ref, deg_ref, b_ref, x1_ref, x2_ref, wl_ref, bl_ref, out_ref):
    H = x1_ref.shape[1]
    dinv = _dinv_block(deg_ref)
    x3 = jnp.maximum(
        jnp.concatenate([acc_ref[0], acc_ref[1]], axis=1) * dinv + b_ref[...],
        0.0)
    p = jnp.dot(x1_ref[...], wl_ref[0:H, :], preferred_element_type=jnp.float32)
    p = p + jnp.dot(x2_ref[...], wl_ref[H:2 * H, :],
                    preferred_element_type=jnp.float32)
    p = p + jnp.dot(x3, wl_ref[2 * H:3 * H, :],
                    preferred_element_type=jnp.float32)
    out_ref[...] = p + bl_ref[...]


def _row_spec(shape2):
    if len(shape2) == 2:
        return pl.BlockSpec((BN, shape2[1]), lambda i: (i, 0))
    return pl.BlockSpec((shape2[0], BN, shape2[2]), lambda i: (0, i, 0))


def _full_spec(shape):
    nd = len(shape)
    return pl.BlockSpec(shape, lambda i: (0,) * nd)


def kernel(x, edge_index, W0, b0, W1, b1, W2, b2, Wlin, blin):
    N, D = x.shape
    H = W0.shape[1]
    OUT = Wlin.shape[1]
    E = edge_index.shape[1]
    F = H // 2
    Etot = E + N

    EPT = -(-Etot // (NS * 4 * CE)) * (4 * CE)
    Epad = NS * EPT
    NACC = -(-(N + 1) // (NS * 8)) * (NS * 8)

    ei = edge_index.astype(jnp.int32)
    loop = jnp.arange(N, dtype=jnp.int32)
    src = jnp.concatenate([ei[0], loop])
    dst = jnp.concatenate([ei[1], loop])
    pad = Epad - Etot
    src_p = jnp.concatenate([src, jnp.zeros((pad,), jnp.int32)])
    dst_p = jnp.concatenate([dst, jnp.full((pad,), N, jnp.int32)])
    src2 = jnp.concatenate([src_p, src_p + N])
    nch = Epad // (NS * CE)
    sd = jnp.stack([
        src2.reshape(NC, NS, nch, CE),
        jnp.broadcast_to(dst_p.reshape(1, NS, nch, CE), (NC, NS, nch, CE)),
    ], axis=3)

    ones16 = jnp.ones((CE, 16), jnp.float32)
    zer_d = jnp.zeros((NACC, 16), jnp.float32)
    zer_f = jnp.zeros((NACC, F), jnp.float32)
    b0r = b0.reshape(1, H)
    b1r = b1.reshape(1, H)
    b2r = b2.reshape(1, H)
    blr = blin.reshape(1, OUT)

    deg_k = _make_deg_kernel(Epad, N, NACC)
    prop_k = _make_prop_kernel(Epad, N, NACC, F)
    grid = (N // BN,)

    deg2 = deg_k(dst_p, ones16, zer_d)

    g0 = pl.pallas_call(
        _k0_body,
        grid=grid,
        in_specs=[_row_spec((N, D)), _full_spec((D, H)),
                  _row_spec((NC, N, 16))],
        out_specs=_row_spec((NC, N, F)),
        out_shape=jax.ShapeDtypeStruct((NC, N, F), jnp.float32),
    )(x, W0, deg2)

    acc1 = prop_k(g0.reshape(NC * N, F), sd, zer_f)

    def layer(acc, b_r, W_next):
        return pl.pallas_call(
            _k12_body,
            grid=grid,
            in_specs=[_row_spec((NC, N, F)), _row_spec((NC, N, 16)),
                      _full_spec((1, H)), _full_spec((H, H))],
            out_specs=[_row_spec((N, H)), _row_spec((NC, N, F))],
            out_shape=[jax.ShapeDtypeStruct((N, H), jnp.float32),
                       jax.ShapeDtypeStruct((NC, N, F), jnp.float32)],
        )(acc, deg2, b_r, W_next)

    x1, g1 = layer(acc1, b0r, W1)
    acc2 = prop_k(g1.reshape(NC * N, F), sd, zer_f)
    x2, g2 = layer(acc2, b1r, W2)
    acc3 = prop_k(g2.reshape(NC * N, F), sd, zer_f)

    pred = pl.pallas_call(
        _k3_body,
        grid=grid,
        in_specs=[_row_spec((NC, N, F)), _row_spec((NC, N, 16)),
                  _full_spec((1, H)), _row_spec((N, H)), _row_spec((N, H)),
                  _full_spec((3 * H, OUT)), _full_spec((1, OUT))],
        out_specs=_row_spec((N, OUT)),
        out_shape=jax.ShapeDtypeStruct((N, OUT), jnp.float32),
    )(acc3, deg2, b2r, x1, x2, Wlin, blr)

    return pred
